# baseline (speedup 1.0000x reference)
import functools

import jax
import jax.numpy as jnp
from jax import lax
from jax.experimental import pallas as pl
from jax.experimental.pallas import tpu as pltpu
from jax.experimental.pallas import tpu_sc as plsc

N_OP = 10000
N_DEV = 1000
D = 512
DE = 16

_NT = 16
_LANES = 16
_BATCH = 64
_NSUB = 32
_CAP = 5120
_T_OP_ROWS = 2 * N_OP + N_DEV + 1
_T_DEV_ROWS = N_OP + N_DEV + 1
_EF_OP_ROWS = 48000 + 48000 + 19000 + 1
_EF_DEV_ROWS = 19000 + 16000 + 1


def _mm_body(x_ref, w_ref, b_ref, o_ref):
    o_ref[...] = (
        jnp.dot(x_ref[...], w_ref[...], preferred_element_type=jnp.float32)
        + b_ref[...]
    )


def _mm_res_body(x_ref, w_ref, b_ref, r_ref, o_ref):
    o_ref[...] = (
        jnp.dot(x_ref[...], w_ref[...], preferred_element_type=jnp.float32)
        + b_ref[...]
        + r_ref[...]
    )


def _dense(x, w, b, residual=None, bm=512):
    m, k = x.shape
    n = w.shape[1]
    mp = -(-m // bm) * bm
    if mp != m:
        x = jnp.pad(x, ((0, mp - m), (0, 0)))
        if residual is not None:
            residual = jnp.pad(residual, ((0, mp - m), (0, 0)))
    b2 = b.reshape(1, n)
    grid = (mp // bm,)
    x_spec = pl.BlockSpec((bm, k), lambda i: (i, 0))
    w_spec = pl.BlockSpec((k, n), lambda i: (0, 0))
    b_spec = pl.BlockSpec((1, n), lambda i: (0, 0))
    o_spec = pl.BlockSpec((bm, n), lambda i: (i, 0))
    out_shape = jax.ShapeDtypeStruct((mp, n), jnp.float32)
    if residual is None:
        out = pl.pallas_call(
            _mm_body,
            grid=grid,
            in_specs=[x_spec, w_spec, b_spec],
            out_specs=o_spec,
            out_shape=out_shape,
        )(x, w, b2)
    else:
        out = pl.pallas_call(
            _mm_res_body,
            grid=grid,
            in_specs=[x_spec, w_spec, b_spec, o_spec],
            out_specs=o_spec,
            out_shape=out_shape,
        )(x, w, b2, residual)
    return out[:m] if mp != m else out


def _build_stream(dsts, srcs, src_offs, ef_offs, ef_lens, n_dst, zero_t_row,
                  zero_ef_row):
    dst_all = jnp.concatenate(
        list(dsts) + [jnp.arange(n_dst, dtype=jnp.int32)])
    src_all = jnp.concatenate(
        [s + o for s, o in zip(srcs, src_offs)]
        + [jnp.full((n_dst,), zero_t_row, jnp.int32)])
    ef_all = jnp.concatenate(
        [jnp.arange(l, dtype=jnp.int32) + o for l, o in zip(ef_lens, ef_offs)]
        + [jnp.full((n_dst,), zero_ef_row, jnp.int32)])
    order = jnp.argsort(dst_all)
    dst_s = dst_all[order].astype(jnp.int32)
    src_s = src_all[order].astype(jnp.int32)
    ef_s = ef_all[order].astype(jnp.int32)
    e = dst_s.shape[0]
    ideal = (jnp.arange(1, _NSUB, dtype=jnp.int32) * e) // _NSUB
    bdst = dst_s[ideal]
    snapped = jnp.searchsorted(dst_s, bdst, side="left").astype(jnp.int32)
    bounds = jnp.concatenate(
        [jnp.zeros((1,), jnp.int32), snapped,
         jnp.full((1,), e, jnp.int32)])
    dst_s = jnp.pad(dst_s, (0, _CAP), constant_values=n_dst)
    src_s = jnp.pad(src_s, (0, _CAP), constant_values=zero_t_row)
    ef_s = jnp.pad(ef_s, (0, _CAP), constant_values=zero_ef_row)
    idxs = bounds[:-1, None] + jnp.arange(_CAP, dtype=jnp.int32)[None]
    dst_sl = dst_s[idxs].reshape(-1)
    src_sl = src_s[idxs].reshape(-1)
    ef_sl = ef_s[idxs].reshape(-1)
    cnts = jnp.zeros((_NSUB, 128), jnp.int32)
    cnts = cnts.at[:, 0].set(bounds[1:] - bounds[:-1]).reshape(-1)
    return src_sl, dst_sl, ef_sl, cnts


def _stream_pass(sid, trash,
                 src_hbm, dst_hbm, ef_hbm, t_hbm, eft_hbm, out_hbm, bnd_hbm,
                 sidx, didx, eidx, bbuf, trows, efrows, stage, sdid,
                 sem_t, sem_e):
    base = sid * _CAP
    pltpu.sync_copy(bnd_hbm.at[pl.ds(sid * 128, 128)], bbuf)
    cnt = bbuf[pl.ds(0, _LANES)][0]

    pltpu.sync_copy(src_hbm.at[pl.ds(base, _CAP)], sidx.at[pl.ds(0, _CAP)])
    pltpu.sync_copy(dst_hbm.at[pl.ds(base, _CAP)], didx.at[pl.ds(0, _CAP)])
    pltpu.sync_copy(ef_hbm.at[pl.ds(base, _CAP)], eidx.at[pl.ds(0, _CAP)])

    nb = (cnt + (_BATCH - 1)) // _BATCH
    trash_v = jnp.full((_LANES,), trash, jnp.int32)
    lane = lax.iota(jnp.int32, _LANES)

    def batch_body(b, carry):
        off = b * _BATCH
        cp_t = pltpu.async_copy(t_hbm.at[sidx.at[pl.ds(off, _BATCH)]],
                                trows, sem_t)
        cp_e = pltpu.async_copy(eft_hbm.at[eidx.at[pl.ds(off, _BATCH)]],
                                efrows, sem_e)
        cp_t.wait()
        cp_e.wait()

        def group_body(g, carry):
            off_g = off + g * _LANES
            dv = didx[pl.ds(off_g, _LANES)]
            dv = jnp.where(off_g + lane < cnt, dv, trash_v)

            def edge(l, c):
                cur_d, k, s0, s1, s2, s3 = c
                d = dv[l]
                r = g * _LANES + l
                is_new = d != cur_d
                k_pre = k + jnp.where(is_new, jnp.int32(1), jnp.int32(0))
                do_flush = k_pre == _BATCH

                @pl.when(do_flush)
                def _():
                    sdid[pl.ds(0, _LANES)] = s0
                    sdid[pl.ds(_LANES, _LANES)] = s1
                    sdid[pl.ds(2 * _LANES, _LANES)] = s2
                    sdid[pl.ds(3 * _LANES, _LANES)] = s3
                    pltpu.sync_copy(stage, out_hbm.at[sdid])

                kk = jnp.where(do_flush, jnp.int32(0), k_pre)
                ni = jnp.where(is_new, jnp.int32(1), jnp.int32(0))
                dspl = jnp.zeros((_LANES,), jnp.int32) + d
                nv = jnp.zeros((_LANES,), jnp.int32) + ni
                m0 = jnp.where(lane == kk, nv, jnp.zeros((_LANES,), jnp.int32))
                s0 = jnp.where(m0 >= 1, dspl, s0)
                m1 = jnp.where(lane + _LANES == kk, nv,
                               jnp.zeros((_LANES,), jnp.int32))
                s1 = jnp.where(m1 >= 1, dspl, s1)
                m2 = jnp.where(lane + 2 * _LANES == kk, nv,
                               jnp.zeros((_LANES,), jnp.int32))
                s2 = jnp.where(m2 >= 1, dspl, s2)
                m3 = jnp.where(lane + 3 * _LANES == kk, nv,
                               jnp.zeros((_LANES,), jnp.int32))
                s3 = jnp.where(m3 >= 1, dspl, s3)

                @pl.when(is_new)
                def _():
                    def cset(cc, _c):
                        s = pl.ds(cc * _LANES, _LANES)
                        stage[kk, s] = jnp.maximum(
                            trows[r, s] + efrows[r, s], 0.0)
                        return _c

                    lax.fori_loop(0, D // _LANES, cset, jnp.int32(0),
                                  unroll=False)

                @pl.when(jnp.logical_not(is_new))
                def _():
                    def cadd(cc, _c):
                        s = pl.ds(cc * _LANES, _LANES)
                        stage[kk, s] = stage[kk, s] + jnp.maximum(
                            trows[r, s] + efrows[r, s], 0.0)
                        return _c

                    lax.fori_loop(0, D // _LANES, cadd, jnp.int32(0),
                                  unroll=False)

                return (d, kk, s0, s1, s2, s3)

            c = carry
            for l in range(_LANES):
                c = edge(l, c)
            return c

        return lax.fori_loop(0, _BATCH // _LANES, group_body, carry,
                             unroll=False)

    init = (jnp.int32(-1), jnp.int32(-1),
            trash_v, trash_v, trash_v, trash_v)
    cur_d, k, s0, s1, s2, s3 = lax.fori_loop(0, nb, batch_body, init,
                                             unroll=False)

    @pl.when(k >= 0)
    def _():
        sdid[pl.ds(0, _LANES)] = jnp.where(lane <= k, s0, trash_v)
        sdid[pl.ds(_LANES, _LANES)] = jnp.where(lane + _LANES <= k, s1,
                                                trash_v)
        sdid[pl.ds(2 * _LANES, _LANES)] = jnp.where(lane + 2 * _LANES <= k,
                                                    s2, trash_v)
        sdid[pl.ds(3 * _LANES, _LANES)] = jnp.where(lane + 3 * _LANES <= k,
                                                    s3, trash_v)
        pltpu.sync_copy(stage, out_hbm.at[sdid])


def _sc_body(src_op, dst_op, ef_op, bnd_op, t_op, eft_op,
             src_dv, dst_dv, ef_dv, bnd_dv, t_dv, eft_dv,
             op_out, dev_out,
             sidx, didx, eidx, bbuf, trows, efrows, stage, sdid,
             sem_t, sem_e):
    core = lax.axis_index("c")
    tid = lax.axis_index("s")
    sid = core * _NT + tid
    _stream_pass(sid, jnp.int32(N_OP),
                 src_op, dst_op, ef_op, t_op, eft_op, op_out, bnd_op,
                 sidx, didx, eidx, bbuf, trows, efrows, stage, sdid,
                 sem_t, sem_e)
    _stream_pass(sid, jnp.int32(N_DEV),
                 src_dv, dst_dv, ef_dv, t_dv, eft_dv, dev_out, bnd_dv,
                 sidx, didx, eidx, bbuf, trows, efrows, stage, sdid,
                 sem_t, sem_e)


@functools.partial(
    pl.kernel,
    out_type=(
        jax.ShapeDtypeStruct((N_OP + 1, D), jnp.float32),
        jax.ShapeDtypeStruct((N_DEV + 1, D), jnp.float32),
    ),
    mesh=plsc.VectorSubcoreMesh(core_axis_name="c", subcore_axis_name="s"),
    scratch_types=(
        pltpu.VMEM((_CAP,), jnp.int32),
        pltpu.VMEM((_CAP,), jnp.int32),
        pltpu.VMEM((_CAP,), jnp.int32),
        pltpu.VMEM((128,), jnp.int32),
        pltpu.VMEM((_BATCH, D), jnp.float32),
        pltpu.VMEM((_BATCH, D), jnp.float32),
        pltpu.VMEM((_BATCH, D), jnp.float32),
        pltpu.VMEM((_BATCH,), jnp.int32),
        pltpu.SemaphoreType.DMA,
        pltpu.SemaphoreType.DMA,
    ),
)
def _sc_edges(*refs):
    _sc_body(*refs)


def kernel(op_feats, device_feats,
           prev_src, prev_dst, prev_edge_feats, W_prev, b_prev,
           succ_src, succ_dst, succ_edge_feats, W_succ, b_succ,
           place_src, place_dst, place_edge_feats, W_place, b_place,
           serve_src, serve_dst, serve_edge_feats, W_serve, b_serve,
           link_src, link_dst, link_edge_feats, W_link, b_link,
           W_op_final, b_op_final, W_device_final, b_device_final):
    zb = jnp.zeros((D,), jnp.float32)
    zrow = jnp.zeros((1, D), jnp.float32)
    t_prev = _dense(op_feats, W_prev[:D], zb)
    t_succ = _dense(op_feats, W_succ[:D], zb)
    t_place = _dense(op_feats, W_place[:D], zb)
    t_serve = _dense(device_feats, W_serve[:D], zb)
    t_link = _dense(device_feats, W_link[:D], zb)
    ef_prev = _dense(prev_edge_feats, W_prev[D:], b_prev)
    ef_succ = _dense(succ_edge_feats, W_succ[D:], b_succ)
    ef_place = _dense(place_edge_feats, W_place[D:], b_place)
    ef_serve = _dense(serve_edge_feats, W_serve[D:], b_serve)
    ef_link = _dense(link_edge_feats, W_link[D:], b_link)

    t_op = jnp.concatenate([t_prev, t_succ, t_serve, zrow])
    eft_op = jnp.concatenate([ef_prev, ef_succ, ef_serve, zrow])
    t_dv = jnp.concatenate([t_place, t_link, zrow])
    eft_dv = jnp.concatenate([ef_place, ef_link, zrow])

    src_op, dst_op, efid_op, bnd_op = _build_stream(
        (prev_dst, succ_dst, serve_dst),
        (prev_src, succ_src, serve_src),
        (0, N_OP, 2 * N_OP),
        (0, 48000, 96000), (48000, 48000, 19000),
        N_OP, _T_OP_ROWS - 1, _EF_OP_ROWS - 1)
    src_dv, dst_dv, efid_dv, bnd_dv = _build_stream(
        (place_dst, link_dst),
        (place_src, link_src),
        (0, N_OP),
        (0, 19000), (19000, 16000),
        N_DEV, _T_DEV_ROWS - 1, _EF_DEV_ROWS - 1)

    op_acc, dev_acc = _sc_edges(
        src_op, dst_op, efid_op, bnd_op, t_op, eft_op,
        src_dv, dst_dv, efid_dv, bnd_dv, t_dv, eft_dv,
    )

    op_res = _dense(op_acc[:N_OP], W_op_final, b_op_final,
                    residual=op_feats)
    dev_res = _dense(dev_acc[:N_DEV], W_device_final, b_device_final,
                     residual=device_feats)
    return (op_res, dev_res)
